# baseline (speedup 1.0000x reference)
import functools

import jax
import jax.numpy as jnp
from jax import lax
from jax.experimental import pallas as pl
from jax.experimental.pallas import tpu as pltpu
from jax.experimental.pallas import tpu_sc as plsc

N = 2048
DIM = 2048
HID = 1024
E = 8
K = 2
T = 256
FT = 256
NT = 24
NP = NT * FT
NW = 32
TPW = N // NW
CH = 16


def _router_body(x_ref, gw_ref, pos1_ref, pos2_ref, s1_ref, s2_ref, eot_ref):
    x = x_ref[...]
    gw = gw_ref[...]
    logits = lax.dot_general(x, gw, (((1,), (1,)), ((), ())),
                             preferred_element_type=jnp.float32)
    scores = jax.nn.sigmoid(logits)
    iota_e = lax.broadcasted_iota(jnp.int32, (N, E), 1).astype(jnp.float32)

    m1 = jnp.max(scores, axis=1, keepdims=True)
    a1 = jnp.min(jnp.where(scores >= m1, iota_e, 1e9), axis=1, keepdims=True)
    oh1 = iota_e == a1
    masked = jnp.where(oh1, -jnp.inf, scores)
    m2 = jnp.max(masked, axis=1, keepdims=True)
    a2 = jnp.min(jnp.where(masked >= m2, iota_e, 1e9), axis=1, keepdims=True)
    oh2 = iota_e == a2

    denom = m1 + m2 + 1e-20
    s1_ref[...] = m1 / denom
    s2_ref[...] = m2 / denom

    H = oh1.astype(jnp.float32) + oh2.astype(jnp.float32)
    r = lax.broadcasted_iota(jnp.int32, (T, T), 0)
    c = lax.broadcasted_iota(jnp.int32, (T, T), 1)
    tril = (r >= c).astype(jnp.float32)
    tot = jnp.zeros((1, E), jnp.float32)
    blocks = []
    for b in range(N // T):
        hb = lax.slice(H, (b * T, 0), ((b + 1) * T, E))
        pb = lax.dot_general(tril, hb, (((1,), (0,)), ((), ())),
                             preferred_element_type=jnp.float32) + tot
        blocks.append(pb)
        tot = lax.slice(pb, (T - 1, 0), (T, E))
    P = jnp.concatenate(blocks, axis=0)
    counts = tot

    pt_cnt = jnp.floor((counts + (FT - 1)) * (1.0 / FT))
    r8 = lax.broadcasted_iota(jnp.int32, (E, E), 0)
    c8 = lax.broadcasted_iota(jnp.int32, (E, E), 1)
    excl = (r8 < c8).astype(jnp.float32)
    pt_off = lax.dot_general(pt_cnt, excl, (((1,), (0,)), ((), ())),
                             preferred_element_type=jnp.float32)

    base = float(FT) * pt_off + P
    pos1 = jnp.sum(jnp.where(oh1, base, 0.0), axis=1, keepdims=True) - 1.0
    pos2 = jnp.sum(jnp.where(oh2, base, 0.0), axis=1, keepdims=True) - 1.0
    pos1_ref[...] = pos1.astype(jnp.int32)
    pos2_ref[...] = pos2.astype(jnp.int32)

    pt_cum = pt_off + pt_cnt
    jv = lax.broadcasted_iota(jnp.int32, (NT + 8, E), 0).astype(jnp.float32)
    eot = jnp.sum((jv >= pt_cum).astype(jnp.int32), axis=1, keepdims=True)
    used = jnp.sum(pt_cnt).astype(jnp.int32)
    row_i = lax.broadcasted_iota(jnp.int32, (NT + 8, 1), 0)
    eot_ref[...] = jnp.where(row_i == NT, used, jnp.minimum(eot, E - 1))


def _router_call(x, gate_w):
    return pl.pallas_call(
        _router_body,
        out_shape=[
            jax.ShapeDtypeStruct((N, 1), jnp.int32),
            jax.ShapeDtypeStruct((N, 1), jnp.int32),
            jax.ShapeDtypeStruct((N, 1), jnp.float32),
            jax.ShapeDtypeStruct((N, 1), jnp.float32),
            jax.ShapeDtypeStruct((NT + 8, 1), jnp.int32),
        ],
    )(x, gate_w)


def _dispatch_body(x_hbm, pos1_hbm, pos2_hbm, out_hbm, idx1_v, idx2_v, buf,
                   sem_ld, sem_st):
    wid = lax.axis_index("s") * 2 + lax.axis_index("c")
    nch = TPW // CH
    base0 = wid * TPW
    pltpu.sync_copy(x_hbm.at[pl.ds(base0, CH)], buf.at[0])
    pltpu.sync_copy(pos1_hbm.at[pl.ds(base0, CH)], idx1_v.at[0])
    pltpu.sync_copy(pos2_hbm.at[pl.ds(base0, CH)], idx2_v.at[0])
    for c in range(nch):
        b, nb = c % 2, (c + 1) % 2
        lds = ()
        if c + 1 < nch:
            nbase = wid * TPW + (c + 1) * CH
            lds = (
                pltpu.async_copy(x_hbm.at[pl.ds(nbase, CH)], buf.at[nb], sem_ld),
                pltpu.async_copy(pos1_hbm.at[pl.ds(nbase, CH)], idx1_v.at[nb], sem_ld),
                pltpu.async_copy(pos2_hbm.at[pl.ds(nbase, CH)], idx2_v.at[nb], sem_ld),
            )
        st1 = pltpu.async_copy(buf.at[b], out_hbm.at[idx1_v.at[b]], sem_st)
        st2 = pltpu.async_copy(buf.at[b], out_hbm.at[idx2_v.at[b]], sem_st)
        st1.wait()
        st2.wait()
        for ld in lds:
            ld.wait()


HH = HID // 2


def _ffn_body(eot_sref, rows_ref, w1_ref, w3_ref, w2_ref, out_ref):
    @pl.when(pl.program_id(0) < eot_sref[NT])
    def _():
        rows = rows_ref[...]
        h1 = lax.dot_general(rows, w1_ref[0], (((1,), (1,)), ((), ())),
                             preferred_element_type=jnp.float32)
        h3 = lax.dot_general(rows, w3_ref[0], (((1,), (1,)), ((), ())),
                             preferred_element_type=jnp.float32)
        g = h1 * jax.nn.sigmoid(h1) * h3
        out_ref[...] = lax.dot_general(g, w2_ref[0], (((1,), (1,)), ((), ())),
                                       preferred_element_type=jnp.float32)


def _ffn_call(eot, routed, W1, W3, W2):
    grid_spec = pltpu.PrefetchScalarGridSpec(
        num_scalar_prefetch=1,
        grid=(NT,),
        in_specs=[
            pl.BlockSpec((FT, DIM), lambda i, eot: (i, 0)),
            pl.BlockSpec((1, HID, DIM), lambda i, eot: (eot[i], 0, 0)),
            pl.BlockSpec((1, HID, DIM), lambda i, eot: (eot[i], 0, 0)),
            pl.BlockSpec((1, DIM, HID), lambda i, eot: (eot[i], 0, 0)),
        ],
        out_specs=pl.BlockSpec((FT, DIM), lambda i, eot: (i, 0)),
    )
    return pl.pallas_call(
        _ffn_body,
        grid_spec=grid_spec,
        out_shape=jax.ShapeDtypeStruct((NP, DIM), jnp.float32),
    )(eot, routed, W1, W3, W2)


CHC = 8


def _combine_body(routed_hbm, pos1_hbm, pos2_hbm, g1_hbm, g2_hbm,
                  idx1_v, idx2_v, buf1, buf2, sem_g, sem_st):
    wid = lax.axis_index("s") * 2 + lax.axis_index("c")
    nch = TPW // CHC
    for c in range(nch):
        b = c % 2
        base = wid * TPW + c * CHC
        pltpu.sync_copy(pos1_hbm.at[pl.ds(base, CHC)], idx1_v.at[b])
        pltpu.sync_copy(pos2_hbm.at[pl.ds(base, CHC)], idx2_v.at[b])
        g1c = pltpu.async_copy(routed_hbm.at[idx1_v.at[b]], buf1.at[b], sem_g)
        g2c = pltpu.async_copy(routed_hbm.at[idx2_v.at[b]], buf2.at[b], sem_g)
        if c > 0:
            pbase = wid * TPW + (c - 1) * CHC
            w1 = pltpu.async_copy(buf1.at[1 - b], g1_hbm.at[pl.ds(pbase, CHC)], sem_st)
            w2 = pltpu.async_copy(buf2.at[1 - b], g2_hbm.at[pl.ds(pbase, CHC)], sem_st)
            w1.wait()
            w2.wait()
        g1c.wait()
        g2c.wait()
    lbase = wid * TPW + (nch - 1) * CHC
    lb = (nch - 1) % 2
    pltpu.sync_copy(buf1.at[lb], g1_hbm.at[pl.ds(lbase, CHC)])
    pltpu.sync_copy(buf2.at[lb], g2_hbm.at[pl.ds(lbase, CHC)])


def _shared_body(x_ref, ws1_ref, ws3_ref, ws2_ref, out_ref):
    xb = x_ref[...]
    h1 = lax.dot_general(xb, ws1_ref[...], (((1,), (1,)), ((), ())),
                         preferred_element_type=jnp.float32)
    h3 = lax.dot_general(xb, ws3_ref[...], (((1,), (1,)), ((), ())),
                         preferred_element_type=jnp.float32)
    g = h1 * jax.nn.sigmoid(h1) * h3
    out_ref[...] = lax.dot_general(g, ws2_ref[...], (((1,), (1,)), ((), ())),
                                   preferred_element_type=jnp.float32)


def _shared_call(x, Ws1, Ws3, Ws2):
    return pl.pallas_call(
        _shared_body,
        grid=(N // T,),
        in_specs=[
            pl.BlockSpec((T, DIM), lambda i: (i, 0)),
            pl.BlockSpec((HID, DIM), lambda i: (0, 0)),
            pl.BlockSpec((HID, DIM), lambda i: (0, 0)),
            pl.BlockSpec((DIM, HID), lambda i: (0, 0)),
        ],
        out_specs=pl.BlockSpec((T, DIM), lambda i: (i, 0)),
        out_shape=jax.ShapeDtypeStruct((N, DIM), jnp.float32),
    )(x, Ws1, Ws3, Ws2)


def _add_body(shared_ref, g1_ref, g2_ref, s1_ref, s2_ref, out_ref):
    out_ref[...] = (shared_ref[...] + g1_ref[...] * s1_ref[...]
                    + g2_ref[...] * s2_ref[...])


def _add_call(shared, g1, g2, s1, s2):
    return pl.pallas_call(
        _add_body,
        grid=(N // T,),
        in_specs=[
            pl.BlockSpec((T, DIM), lambda i: (i, 0)),
            pl.BlockSpec((T, DIM), lambda i: (i, 0)),
            pl.BlockSpec((T, DIM), lambda i: (i, 0)),
            pl.BlockSpec((T, 1), lambda i: (i, 0)),
            pl.BlockSpec((T, 1), lambda i: (i, 0)),
        ],
        out_specs=pl.BlockSpec((T, DIM), lambda i: (i, 0)),
        out_shape=jax.ShapeDtypeStruct((N, DIM), jnp.float32),
    )(shared, g1, g2, s1, s2)


@functools.cache
def _sc_kernels():
    mesh = plsc.VectorSubcoreMesh(core_axis_name="c", subcore_axis_name="s")
    dispatch = pl.kernel(
        _dispatch_body,
        out_type=jax.ShapeDtypeStruct((NP, DIM), jnp.float32),
        mesh=mesh,
        scratch_types=[
            pltpu.VMEM((2, CH), jnp.int32),
            pltpu.VMEM((2, CH), jnp.int32),
            pltpu.VMEM((2, CH, DIM), jnp.float32),
            pltpu.SemaphoreType.DMA,
            pltpu.SemaphoreType.DMA,
        ],
    )
    combine = pl.kernel(
        _combine_body,
        out_type=[
            jax.ShapeDtypeStruct((N, DIM), jnp.float32),
            jax.ShapeDtypeStruct((N, DIM), jnp.float32),
        ],
        mesh=mesh,
        scratch_types=[
            pltpu.VMEM((2, CHC), jnp.int32),
            pltpu.VMEM((2, CHC), jnp.int32),
            pltpu.VMEM((2, CHC, DIM), jnp.float32),
            pltpu.VMEM((2, CHC, DIM), jnp.float32),
            pltpu.SemaphoreType.DMA,
            pltpu.SemaphoreType.DMA,
        ],
    )
    return dispatch, combine


def kernel(x, gate_w, W1, W2, W3, Ws1, Ws2, Ws3):
    dispatch, combine = _sc_kernels()
    pos1, pos2, s1, s2, eot = _router_call(x, gate_w)
    pos1f = pos1.reshape(N)
    pos2f = pos2.reshape(N)
    shared = _shared_call(x, Ws1, Ws3, Ws2)
    routed = dispatch(x, pos1f, pos2f)
    routed_out = _ffn_call(eot.reshape(NT + 8), routed, W1, W3, W2)
    g1, g2 = combine(routed_out, pos1f, pos2f)
    return _add_call(shared, g1, g2, s1, s2)


# speedup vs baseline: 1.0047x; 1.0047x over previous
import functools

import jax
import jax.numpy as jnp
from jax import lax
from jax.experimental import pallas as pl
from jax.experimental.pallas import tpu as pltpu
from jax.experimental.pallas import tpu_sc as plsc

N = 2048
DIM = 2048
HID = 1024
E = 8
K = 2
T = 256
FT = 256
NT = 24
NP = NT * FT
NW = 32
TPW = N // NW
CH = 16


def _router_body(x_ref, gw_ref, pos1_ref, pos2_ref, s1_ref, s2_ref, eot_ref):
    x = x_ref[...]
    gw = gw_ref[...]
    logits = lax.dot_general(x, gw, (((1,), (1,)), ((), ())),
                             preferred_element_type=jnp.float32)
    scores = jax.nn.sigmoid(logits)
    iota_e = lax.broadcasted_iota(jnp.int32, (N, E), 1).astype(jnp.float32)

    m1 = jnp.max(scores, axis=1, keepdims=True)
    a1 = jnp.min(jnp.where(scores >= m1, iota_e, 1e9), axis=1, keepdims=True)
    oh1 = iota_e == a1
    masked = jnp.where(oh1, -jnp.inf, scores)
    m2 = jnp.max(masked, axis=1, keepdims=True)
    a2 = jnp.min(jnp.where(masked >= m2, iota_e, 1e9), axis=1, keepdims=True)
    oh2 = iota_e == a2

    denom = m1 + m2 + 1e-20
    s1_ref[...] = m1 / denom
    s2_ref[...] = m2 / denom

    H = oh1.astype(jnp.float32) + oh2.astype(jnp.float32)
    r = lax.broadcasted_iota(jnp.int32, (T, T), 0)
    c = lax.broadcasted_iota(jnp.int32, (T, T), 1)
    tril = (r >= c).astype(jnp.float32)
    tot = jnp.zeros((1, E), jnp.float32)
    blocks = []
    for b in range(N // T):
        hb = lax.slice(H, (b * T, 0), ((b + 1) * T, E))
        pb = lax.dot_general(tril, hb, (((1,), (0,)), ((), ())),
                             preferred_element_type=jnp.float32) + tot
        blocks.append(pb)
        tot = lax.slice(pb, (T - 1, 0), (T, E))
    P = jnp.concatenate(blocks, axis=0)
    counts = tot

    pt_cnt = jnp.floor((counts + (FT - 1)) * (1.0 / FT))
    r8 = lax.broadcasted_iota(jnp.int32, (E, E), 0)
    c8 = lax.broadcasted_iota(jnp.int32, (E, E), 1)
    excl = (r8 < c8).astype(jnp.float32)
    pt_off = lax.dot_general(pt_cnt, excl, (((1,), (0,)), ((), ())),
                             preferred_element_type=jnp.float32)

    base = float(FT) * pt_off + P
    pos1 = jnp.sum(jnp.where(oh1, base, 0.0), axis=1, keepdims=True) - 1.0
    pos2 = jnp.sum(jnp.where(oh2, base, 0.0), axis=1, keepdims=True) - 1.0
    pos1_ref[...] = pos1.astype(jnp.int32)
    pos2_ref[...] = pos2.astype(jnp.int32)

    pt_cum = pt_off + pt_cnt
    jv = lax.broadcasted_iota(jnp.int32, (NT + 8, E), 0).astype(jnp.float32)
    eot = jnp.sum((jv >= pt_cum).astype(jnp.int32), axis=1, keepdims=True)
    used = jnp.sum(pt_cnt).astype(jnp.int32)
    row_i = lax.broadcasted_iota(jnp.int32, (NT + 8, 1), 0)
    eot_ref[...] = jnp.where(row_i == NT, used, jnp.minimum(eot, E - 1))


def _router_call(x, gate_w):
    return pl.pallas_call(
        _router_body,
        out_shape=[
            jax.ShapeDtypeStruct((N, 1), jnp.int32),
            jax.ShapeDtypeStruct((N, 1), jnp.int32),
            jax.ShapeDtypeStruct((N, 1), jnp.float32),
            jax.ShapeDtypeStruct((N, 1), jnp.float32),
            jax.ShapeDtypeStruct((NT + 8, 1), jnp.int32),
        ],
    )(x, gate_w)


def _dispatch_body(x_hbm, pos1_hbm, pos2_hbm, out_hbm, idx1_v, idx2_v, buf,
                   sem_ld, sem_st):
    wid = lax.axis_index("s") * 2 + lax.axis_index("c")
    nch = TPW // CH
    base0 = wid * TPW
    pltpu.sync_copy(x_hbm.at[pl.ds(base0, CH)], buf.at[0])
    pltpu.sync_copy(pos1_hbm.at[pl.ds(base0, CH)], idx1_v.at[0])
    pltpu.sync_copy(pos2_hbm.at[pl.ds(base0, CH)], idx2_v.at[0])
    for c in range(nch):
        b, nb = c % 2, (c + 1) % 2
        lds = ()
        if c + 1 < nch:
            nbase = wid * TPW + (c + 1) * CH
            lds = (
                pltpu.async_copy(x_hbm.at[pl.ds(nbase, CH)], buf.at[nb], sem_ld),
                pltpu.async_copy(pos1_hbm.at[pl.ds(nbase, CH)], idx1_v.at[nb], sem_ld),
                pltpu.async_copy(pos2_hbm.at[pl.ds(nbase, CH)], idx2_v.at[nb], sem_ld),
            )
        st1 = pltpu.async_copy(buf.at[b], out_hbm.at[idx1_v.at[b]], sem_st)
        st2 = pltpu.async_copy(buf.at[b], out_hbm.at[idx2_v.at[b]], sem_st)
        st1.wait()
        st2.wait()
        for ld in lds:
            ld.wait()


def _ffn_body(eot_sref, rows_ref, w1_ref, w3_ref, w2_ref, out_ref):
    @pl.when(pl.program_id(0) < eot_sref[NT])
    def _():
        rows = rows_ref[...]
        h1 = lax.dot_general(rows, w1_ref[0], (((1,), (1,)), ((), ())),
                             preferred_element_type=jnp.float32)
        h3 = lax.dot_general(rows, w3_ref[0], (((1,), (1,)), ((), ())),
                             preferred_element_type=jnp.float32)
        g = h1 * jax.nn.sigmoid(h1) * h3
        out_ref[...] = lax.dot_general(g, w2_ref[0], (((1,), (1,)), ((), ())),
                                       preferred_element_type=jnp.float32)


def _ffn_call(eot, routed, W1, W3, W2):
    grid_spec = pltpu.PrefetchScalarGridSpec(
        num_scalar_prefetch=1,
        grid=(NT,),
        in_specs=[
            pl.BlockSpec((FT, DIM), lambda i, eot: (i, 0)),
            pl.BlockSpec((1, HID, DIM), lambda i, eot: (eot[i], 0, 0)),
            pl.BlockSpec((1, HID, DIM), lambda i, eot: (eot[i], 0, 0)),
            pl.BlockSpec((1, DIM, HID), lambda i, eot: (eot[i], 0, 0)),
        ],
        out_specs=pl.BlockSpec((FT, DIM), lambda i, eot: (i, 0)),
    )
    return pl.pallas_call(
        _ffn_body,
        grid_spec=grid_spec,
        out_shape=jax.ShapeDtypeStruct((NP, DIM), jnp.float32),
    )(eot, routed, W1, W3, W2)


CHC = 8


def _combine_body(routed_hbm, pos1_hbm, pos2_hbm, g1_hbm, g2_hbm,
                  idx1_v, idx2_v, buf1, buf2, sem_g, sem_st):
    wid = lax.axis_index("s") * 2 + lax.axis_index("c")
    nch = TPW // CHC
    for c in range(nch):
        b = c % 2
        base = wid * TPW + c * CHC
        pltpu.sync_copy(pos1_hbm.at[pl.ds(base, CHC)], idx1_v.at[b])
        pltpu.sync_copy(pos2_hbm.at[pl.ds(base, CHC)], idx2_v.at[b])
        g1c = pltpu.async_copy(routed_hbm.at[idx1_v.at[b]], buf1.at[b], sem_g)
        g2c = pltpu.async_copy(routed_hbm.at[idx2_v.at[b]], buf2.at[b], sem_g)
        if c > 0:
            pbase = wid * TPW + (c - 1) * CHC
            w1 = pltpu.async_copy(buf1.at[1 - b], g1_hbm.at[pl.ds(pbase, CHC)], sem_st)
            w2 = pltpu.async_copy(buf2.at[1 - b], g2_hbm.at[pl.ds(pbase, CHC)], sem_st)
            w1.wait()
            w2.wait()
        g1c.wait()
        g2c.wait()
    lbase = wid * TPW + (nch - 1) * CHC
    lb = (nch - 1) % 2
    pltpu.sync_copy(buf1.at[lb], g1_hbm.at[pl.ds(lbase, CHC)])
    pltpu.sync_copy(buf2.at[lb], g2_hbm.at[pl.ds(lbase, CHC)])


def _shared_body(x_ref, ws1_ref, ws3_ref, ws2_ref, out_ref):
    xb = x_ref[...]
    h1 = lax.dot_general(xb, ws1_ref[...], (((1,), (1,)), ((), ())),
                         preferred_element_type=jnp.float32)
    h3 = lax.dot_general(xb, ws3_ref[...], (((1,), (1,)), ((), ())),
                         preferred_element_type=jnp.float32)
    g = h1 * jax.nn.sigmoid(h1) * h3
    out_ref[...] = lax.dot_general(g, ws2_ref[...], (((1,), (1,)), ((), ())),
                                   preferred_element_type=jnp.float32)


def _shared_call(x, Ws1, Ws3, Ws2):
    return pl.pallas_call(
        _shared_body,
        grid=(N // T,),
        in_specs=[
            pl.BlockSpec((T, DIM), lambda i: (i, 0)),
            pl.BlockSpec((HID, DIM), lambda i: (0, 0)),
            pl.BlockSpec((HID, DIM), lambda i: (0, 0)),
            pl.BlockSpec((DIM, HID), lambda i: (0, 0)),
        ],
        out_specs=pl.BlockSpec((T, DIM), lambda i: (i, 0)),
        out_shape=jax.ShapeDtypeStruct((N, DIM), jnp.float32),
    )(x, Ws1, Ws3, Ws2)


def _add_body(shared_ref, g1_ref, g2_ref, s1_ref, s2_ref, out_ref):
    out_ref[...] = (shared_ref[...] + g1_ref[...] * s1_ref[...]
                    + g2_ref[...] * s2_ref[...])


def _add_call(shared, g1, g2, s1, s2):
    return pl.pallas_call(
        _add_body,
        grid=(N // T,),
        in_specs=[
            pl.BlockSpec((T, DIM), lambda i: (i, 0)),
            pl.BlockSpec((T, DIM), lambda i: (i, 0)),
            pl.BlockSpec((T, DIM), lambda i: (i, 0)),
            pl.BlockSpec((T, 1), lambda i: (i, 0)),
            pl.BlockSpec((T, 1), lambda i: (i, 0)),
        ],
        out_specs=pl.BlockSpec((T, DIM), lambda i: (i, 0)),
        out_shape=jax.ShapeDtypeStruct((N, DIM), jnp.float32),
    )(shared, g1, g2, s1, s2)


@functools.cache
def _sc_kernels():
    mesh = plsc.VectorSubcoreMesh(core_axis_name="c", subcore_axis_name="s")
    dispatch = pl.kernel(
        _dispatch_body,
        out_type=jax.ShapeDtypeStruct((NP, DIM), jnp.float32),
        mesh=mesh,
        scratch_types=[
            pltpu.VMEM((2, CH), jnp.int32),
            pltpu.VMEM((2, CH), jnp.int32),
            pltpu.VMEM((2, CH, DIM), jnp.float32),
            pltpu.SemaphoreType.DMA,
            pltpu.SemaphoreType.DMA,
        ],
    )
    combine = pl.kernel(
        _combine_body,
        out_type=[
            jax.ShapeDtypeStruct((N, DIM), jnp.float32),
            jax.ShapeDtypeStruct((N, DIM), jnp.float32),
        ],
        mesh=mesh,
        scratch_types=[
            pltpu.VMEM((2, CHC), jnp.int32),
            pltpu.VMEM((2, CHC), jnp.int32),
            pltpu.VMEM((2, CHC, DIM), jnp.float32),
            pltpu.VMEM((2, CHC, DIM), jnp.float32),
            pltpu.SemaphoreType.DMA,
            pltpu.SemaphoreType.DMA,
        ],
    )
    return dispatch, combine


def kernel(x, gate_w, W1, W2, W3, Ws1, Ws2, Ws3):
    dispatch, combine = _sc_kernels()
    pos1, pos2, s1, s2, eot = _router_call(x, gate_w)
    pos1f = pos1.reshape(N)
    pos2f = pos2.reshape(N)
    shared = _shared_call(x, Ws1, Ws3, Ws2)
    routed = dispatch(x, pos1f, pos2f)
    routed_out = _ffn_call(eot.reshape(NT + 8), routed, W1, W3, W2)
    g1, g2 = combine(routed_out, pos1f, pos2f)
    return _add_call(shared, g1, g2, s1, s2)
